# baseline (speedup 1.0000x reference)
import jax
import jax.numpy as jnp
from jax.experimental import pallas as pl
from jax.experimental.pallas import tpu as pltpu

_B, _K, _D = 64, 1, 1024
_HS, _HR, _E = 4096, 1024, 8
_S = 4
_G = _E * _S
_CS = _HS // _G
_CR = _HR // _S


def _step(x_ref, rw_ref, wg_s_ref, bg_s_ref, wu_s_ref, bu_s_ref, wd_s_ref,
          bd_s_ref, wg_r_ref, bg_r_ref, wu_r_ref, bu_r_ref, wd_r_ref,
          bd_r_ref, out_ref):
    i = pl.program_id(0)
    j = i % _S
    xv = x_ref[...]

    g = jnp.dot(xv, wg_s_ref[...], preferred_element_type=jnp.float32)
    u = jnp.dot(xv, wu_s_ref[...], preferred_element_type=jnp.float32)
    h = jax.nn.silu(g + bg_s_ref[...]) * (u + bu_s_ref[...])
    acc = jnp.dot(h, wd_s_ref[...], preferred_element_type=jnp.float32)

    w = rw_ref[0]
    gr = jnp.dot(xv, wg_r_ref[0], preferred_element_type=jnp.float32)
    ur = jnp.dot(xv, wu_r_ref[0], preferred_element_type=jnp.float32)
    hr = jax.nn.silu(gr + bg_r_ref[0]) * (ur + bu_r_ref[0]) * w
    acc = acc + jnp.dot(hr, wd_r_ref[0], preferred_element_type=jnp.float32)
    acc = acc + jnp.where(j == 0, 1.0, 0.0) * (w * bd_r_ref[0])

    @pl.when(i == 0)
    def _init():
        out_ref[...] = acc + bd_s_ref[...]

    @pl.when(i != 0)
    def _accum():
        out_ref[...] += acc


def kernel(x, routing_weights, Wg_s, bg_s, Wu_s, bu_s, Wd_s, bd_s,
           Wg_r, bg_r, Wu_r, bu_r, Wd_r, bd_r):
    x2 = x.reshape(_B, _D)
    rw = routing_weights.T.reshape(_E, _B, 1)
    bg_r3 = bg_r.reshape(_E, 1, _HR)
    bu_r3 = bu_r.reshape(_E, 1, _HR)
    bd_r3 = bd_r.reshape(_E, 1, _D)

    out = pl.pallas_call(
        _step,
        grid=(_G,),
        in_specs=[
            pl.BlockSpec((_B, _D), lambda i: (0, 0)),
            pl.BlockSpec((1, _B, 1), lambda i: (i // _S, 0, 0)),
            pl.BlockSpec((_D, _CS), lambda i: (0, i)),
            pl.BlockSpec((_CS,), lambda i: (i,)),
            pl.BlockSpec((_D, _CS), lambda i: (0, i)),
            pl.BlockSpec((_CS,), lambda i: (i,)),
            pl.BlockSpec((_CS, _D), lambda i: (i, 0)),
            pl.BlockSpec((_D,), lambda i: (0,)),
            pl.BlockSpec((1, _D, _CR), lambda i: (i // _S, 0, i % _S)),
            pl.BlockSpec((1, 1, _CR), lambda i: (i // _S, 0, i % _S)),
            pl.BlockSpec((1, _D, _CR), lambda i: (i // _S, 0, i % _S)),
            pl.BlockSpec((1, 1, _CR), lambda i: (i // _S, 0, i % _S)),
            pl.BlockSpec((1, _CR, _D), lambda i: (i // _S, i % _S, 0)),
            pl.BlockSpec((1, 1, _D), lambda i: (i // _S, 0, 0)),
        ],
        out_specs=pl.BlockSpec((_B, _D), lambda i: (0, 0)),
        out_shape=jax.ShapeDtypeStruct((_B, _D), jnp.float32),
        compiler_params=pltpu.CompilerParams(
            dimension_semantics=("arbitrary",),
        ),
    )(x2, rw, Wg_s, bg_s, Wu_s, bu_s, Wd_s, bd_s,
      Wg_r, bg_r3, Wu_r, bu_r3, Wd_r, bd_r3)

    return out.reshape(_B, _K, _D)


# speedup vs baseline: 1.1379x; 1.1379x over previous
import jax
import jax.numpy as jnp
from jax.experimental import pallas as pl
from jax.experimental.pallas import tpu as pltpu

_B, _K, _D = 64, 1, 1024
_HS, _HR, _E = 4096, 1024, 8
_S = 2
_G = _E * _S
_CS = _HS // _G
_CR = _HR // _S


def _step(x_ref, rw_ref, wg_s_ref, bg_s_ref, wu_s_ref, bu_s_ref, wd_s_ref,
          bd_s_ref, wg_r_ref, bg_r_ref, wu_r_ref, bu_r_ref, wd_r_ref,
          bd_r_ref, out_ref):
    i = pl.program_id(0)
    j = i % _S
    xv = x_ref[...]

    g = jnp.dot(xv, wg_s_ref[...], preferred_element_type=jnp.float32)
    u = jnp.dot(xv, wu_s_ref[...], preferred_element_type=jnp.float32)
    h = jax.nn.silu(g + bg_s_ref[...]) * (u + bu_s_ref[...])
    acc = jnp.dot(h, wd_s_ref[...], preferred_element_type=jnp.float32)

    w = rw_ref[0]
    gr = jnp.dot(xv, wg_r_ref[0], preferred_element_type=jnp.float32)
    ur = jnp.dot(xv, wu_r_ref[0], preferred_element_type=jnp.float32)
    hr = jax.nn.silu(gr + bg_r_ref[0]) * (ur + bu_r_ref[0]) * w
    acc = acc + jnp.dot(hr, wd_r_ref[0], preferred_element_type=jnp.float32)
    acc = acc + jnp.where(j == 0, 1.0, 0.0) * (w * bd_r_ref[0])

    @pl.when(i == 0)
    def _init():
        out_ref[...] = acc + bd_s_ref[...]

    @pl.when(i != 0)
    def _accum():
        out_ref[...] += acc


def kernel(x, routing_weights, Wg_s, bg_s, Wu_s, bu_s, Wd_s, bd_s,
           Wg_r, bg_r, Wu_r, bu_r, Wd_r, bd_r):
    x2 = x.reshape(_B, _D)
    rw = routing_weights.T.reshape(_E, _B, 1)
    bg_r3 = bg_r.reshape(_E, 1, _HR)
    bu_r3 = bu_r.reshape(_E, 1, _HR)
    bd_r3 = bd_r.reshape(_E, 1, _D)

    out = pl.pallas_call(
        _step,
        grid=(_G,),
        in_specs=[
            pl.BlockSpec((_B, _D), lambda i: (0, 0)),
            pl.BlockSpec((1, _B, 1), lambda i: (i // _S, 0, 0)),
            pl.BlockSpec((_D, _CS), lambda i: (0, i)),
            pl.BlockSpec((_CS,), lambda i: (i,)),
            pl.BlockSpec((_D, _CS), lambda i: (0, i)),
            pl.BlockSpec((_CS,), lambda i: (i,)),
            pl.BlockSpec((_CS, _D), lambda i: (i, 0)),
            pl.BlockSpec((_D,), lambda i: (0,)),
            pl.BlockSpec((1, _D, _CR), lambda i: (i // _S, 0, i % _S)),
            pl.BlockSpec((1, 1, _CR), lambda i: (i // _S, 0, i % _S)),
            pl.BlockSpec((1, _D, _CR), lambda i: (i // _S, 0, i % _S)),
            pl.BlockSpec((1, 1, _CR), lambda i: (i // _S, 0, i % _S)),
            pl.BlockSpec((1, _CR, _D), lambda i: (i // _S, i % _S, 0)),
            pl.BlockSpec((1, 1, _D), lambda i: (i // _S, 0, 0)),
        ],
        out_specs=pl.BlockSpec((_B, _D), lambda i: (0, 0)),
        out_shape=jax.ShapeDtypeStruct((_B, _D), jnp.float32),
        compiler_params=pltpu.CompilerParams(
            dimension_semantics=("arbitrary",),
        ),
    )(x2, rw, Wg_s, bg_s, Wu_s, bu_s, Wd_s, bd_s,
      Wg_r, bg_r3, Wu_r, bu_r3, Wd_r, bd_r3)

    return out.reshape(_B, _K, _D)
